# baseline (speedup 1.0000x reference)
import functools

import jax
import jax.numpy as jnp
from jax import lax
from jax.experimental import pallas as pl
from jax.experimental.pallas import tpu as pltpu
from jax.experimental.pallas import tpu_sc as plsc

_B, _T, _DIM, _CDIM, _K = 16, 2048, 512, 64, 1024
_ROWS = _B * _T
_RB = 512
_NBLK = _ROWS // _RB


def _argmax_body(x_ref, wint_ref, bin_ref, embt_ref, ind_ref):
    z = jnp.dot(x_ref[...], wint_ref[...],
                preferred_element_type=jnp.float32) + bin_ref[...]
    et = embt_ref[...]
    esq = jnp.sum(et * et, axis=0, keepdims=True)
    fe = jnp.dot(z, et, preferred_element_type=jnp.float32)
    flatsq = jnp.sum(z * z, axis=1, keepdims=True)
    dist = -(flatsq - 2.0 * fe + esq)
    ind_ref[0, 0, :] = jnp.argmax(dist, axis=1).astype(jnp.int32)


def _compute_indices(x2d, w_in_t, b_in2d, emb_t):
    ind3 = pl.pallas_call(
        _argmax_body,
        grid=(_NBLK,),
        in_specs=[
            pl.BlockSpec((_RB, _DIM), lambda i: (i, 0)),
            pl.BlockSpec((_DIM, _CDIM), lambda i: (0, 0)),
            pl.BlockSpec((1, _CDIM), lambda i: (0, 0)),
            pl.BlockSpec((_CDIM, _K), lambda i: (0, 0)),
        ],
        out_specs=pl.BlockSpec((1, 1, _RB), lambda i: (i, 0, 0)),
        out_shape=jax.ShapeDtypeStruct((_NBLK, 1, _RB), jnp.int32),
        compiler_params=pltpu.CompilerParams(
            dimension_semantics=("arbitrary",)),
    )(x2d, w_in_t, b_in2d, emb_t)
    return ind3.reshape(_ROWS)


def _table_body(emb_ref, woutt_ref, bout_ref, table_ref):
    table_ref[...] = jnp.dot(emb_ref[...], woutt_ref[...],
                             preferred_element_type=jnp.float32) + bout_ref[...]


def _compute_table(embed, w_out_t, b_out2d):
    return pl.pallas_call(
        _table_body,
        out_shape=jax.ShapeDtypeStruct((_K, _DIM), jnp.float32),
    )(embed, w_out_t, b_out2d)


_NC = 2
_NS = 16
_NW = _NC * _NS
_BPW = _ROWS // _NW
_CH = 64
_NCH = _BPW // _CH


def _sc_gather_body(table_hbm, idx_hbm, out_hbm, idx_v, buf0, buf1, g0, g1):
    wid = lax.axis_index("s") * _NC + lax.axis_index("c")
    base = wid * _BPW
    pltpu.sync_copy(idx_hbm.at[pl.ds(base, _BPW)], idx_v)
    pltpu.async_copy(table_hbm.at[idx_v.at[pl.ds(0, _CH)]], buf0, g0)

    def body(i, _):
        c0 = 2 * i
        pltpu.async_copy(
            table_hbm.at[idx_v.at[pl.ds((c0 + 1) * _CH, _CH)]], buf1, g1)
        pltpu.make_async_copy(
            table_hbm.at[idx_v.at[pl.ds(c0 * _CH, _CH)]], buf0, g0).wait()
        pltpu.sync_copy(buf0, out_hbm.at[pl.ds(base + c0 * _CH, _CH)])

        @pl.when(i < _NCH // 2 - 1)
        def _():
            pltpu.async_copy(
                table_hbm.at[idx_v.at[pl.ds((c0 + 2) * _CH, _CH)]], buf0, g0)

        pltpu.make_async_copy(
            table_hbm.at[idx_v.at[pl.ds((c0 + 1) * _CH, _CH)]], buf1, g1).wait()
        pltpu.sync_copy(buf1, out_hbm.at[pl.ds(base + (c0 + 1) * _CH, _CH)])
        return 0

    lax.fori_loop(0, _NCH // 2, body, 0)


def _sc_gather(table, ind):
    mesh = plsc.VectorSubcoreMesh(core_axis_name="c", subcore_axis_name="s")
    k = functools.partial(
        pl.kernel,
        mesh=mesh,
        out_type=jax.ShapeDtypeStruct((_ROWS, _DIM), jnp.float32),
        scratch_types=[
            pltpu.VMEM((_BPW,), jnp.int32),
            pltpu.VMEM((_CH, _DIM), jnp.float32),
            pltpu.VMEM((_CH, _DIM), jnp.float32),
            pltpu.SemaphoreType.DMA,
            pltpu.SemaphoreType.DMA,
        ],
    )(_sc_gather_body)
    return k(table, ind)


def kernel(x, W_in, b_in, W_out, b_out, embed):
    x2d = x.reshape(_ROWS, _DIM)
    w_in_t = W_in.T
    emb_t = embed.T
    b_in2d = b_in.reshape(1, _CDIM)
    w_out_t = W_out.T
    b_out2d = b_out.reshape(1, _DIM)

    ind = _compute_indices(x2d, w_in_t, b_in2d, emb_t)
    table = _compute_table(embed, w_out_t, b_out2d)
    out = _sc_gather(table, ind)
    return out.reshape(_B, _T, _DIM)


# speedup vs baseline: 1.0074x; 1.0074x over previous
import functools

import jax
import jax.numpy as jnp
from jax import lax
from jax.experimental import pallas as pl
from jax.experimental.pallas import tpu as pltpu
from jax.experimental.pallas import tpu_sc as plsc

_B, _T, _DIM, _CDIM, _K = 16, 2048, 512, 64, 1024
_CPAD = 128
_ROWS = _B * _T
_RB = 512
_NBLK = _ROWS // _RB


def _argmax_body(x_ref, wint_ref, bin_ref, embt_ref, ind_ref):
    z = jnp.dot(x_ref[...], wint_ref[...],
                preferred_element_type=jnp.float32) + bin_ref[...]
    et = embt_ref[...]
    esq = jnp.sum(et * et, axis=0, keepdims=True)
    fe = jnp.dot(z, et, preferred_element_type=jnp.float32)
    flatsq = jnp.sum(z * z, axis=1, keepdims=True)
    dist = -(flatsq - 2.0 * fe + esq)
    ind_ref[0, 0, :] = jnp.argmax(dist, axis=1).astype(jnp.int32)


def _compute_indices(x2d, w_in_t, b_in2d, emb_t):
    ind3 = pl.pallas_call(
        _argmax_body,
        grid=(_NBLK,),
        in_specs=[
            pl.BlockSpec((_RB, _DIM), lambda i: (i, 0)),
            pl.BlockSpec((_DIM, _CDIM), lambda i: (0, 0)),
            pl.BlockSpec((1, _CDIM), lambda i: (0, 0)),
            pl.BlockSpec((_CDIM, _K), lambda i: (0, 0)),
        ],
        out_specs=pl.BlockSpec((1, 1, _RB), lambda i: (i, 0, 0)),
        out_shape=jax.ShapeDtypeStruct((_NBLK, 1, _RB), jnp.int32),
        compiler_params=pltpu.CompilerParams(
            dimension_semantics=("arbitrary",)),
    )(x2d, w_in_t, b_in2d, emb_t)
    return ind3.reshape(_ROWS)


_NC = 2
_NS = 16
_NW = _NC * _NS
_BPW = _ROWS // _NW
_CH = 128
_NCH = _BPW // _CH


def _sc_gather_body(emb_hbm, idx_hbm, q_hbm, idx_v, buf0, buf1, g0, g1):
    wid = lax.axis_index("s") * _NC + lax.axis_index("c")
    base = wid * _BPW
    pltpu.sync_copy(idx_hbm.at[pl.ds(base, _BPW)], idx_v)
    pltpu.async_copy(emb_hbm.at[idx_v.at[pl.ds(0, _CH)]], buf0, g0)

    def body(i, _):
        c0 = 2 * i
        pltpu.async_copy(
            emb_hbm.at[idx_v.at[pl.ds((c0 + 1) * _CH, _CH)]], buf1, g1)
        pltpu.make_async_copy(
            emb_hbm.at[idx_v.at[pl.ds(c0 * _CH, _CH)]], buf0, g0).wait()
        pltpu.sync_copy(buf0, q_hbm.at[pl.ds(base + c0 * _CH, _CH)])

        @pl.when(i < _NCH // 2 - 1)
        def _():
            pltpu.async_copy(
                emb_hbm.at[idx_v.at[pl.ds((c0 + 2) * _CH, _CH)]], buf0, g0)

        pltpu.make_async_copy(
            emb_hbm.at[idx_v.at[pl.ds((c0 + 1) * _CH, _CH)]], buf1, g1).wait()
        pltpu.sync_copy(buf1, q_hbm.at[pl.ds(base + (c0 + 1) * _CH, _CH)])
        return 0

    lax.fori_loop(0, _NCH // 2, body, 0)


def _sc_gather(embed_pad, ind):
    mesh = plsc.VectorSubcoreMesh(core_axis_name="c", subcore_axis_name="s")
    k = functools.partial(
        pl.kernel,
        mesh=mesh,
        out_type=jax.ShapeDtypeStruct((_ROWS, _CPAD), jnp.float32),
        scratch_types=[
            pltpu.VMEM((_BPW,), jnp.int32),
            pltpu.VMEM((_CH, _CPAD), jnp.float32),
            pltpu.VMEM((_CH, _CPAD), jnp.float32),
            pltpu.SemaphoreType.DMA,
            pltpu.SemaphoreType.DMA,
        ],
    )(_sc_gather_body)
    return k(embed_pad, ind)


def _proj_out_body(q_ref, woutt_ref, bout_ref, out_ref):
    out_ref[...] = jnp.dot(q_ref[...], woutt_ref[...],
                           preferred_element_type=jnp.float32) + bout_ref[...]


def _project_out(q, w_out_t, b_out2d):
    return pl.pallas_call(
        _proj_out_body,
        grid=(_NBLK,),
        in_specs=[
            pl.BlockSpec((_RB, _CPAD), lambda i: (i, 0)),
            pl.BlockSpec((_CPAD, _DIM), lambda i: (0, 0)),
            pl.BlockSpec((1, _DIM), lambda i: (0, 0)),
        ],
        out_specs=pl.BlockSpec((_RB, _DIM), lambda i: (i, 0)),
        out_shape=jax.ShapeDtypeStruct((_ROWS, _DIM), jnp.float32),
        compiler_params=pltpu.CompilerParams(
            dimension_semantics=("arbitrary",)),
    )(q, w_out_t, b_out2d)


def kernel(x, W_in, b_in, W_out, b_out, embed):
    x2d = x.reshape(_ROWS, _DIM)
    w_in_t = W_in.T
    emb_t = embed.T
    b_in2d = b_in.reshape(1, _CDIM)
    w_out_t = W_out.T
    b_out2d = b_out.reshape(1, _DIM)

    embed_pad = jnp.pad(embed, ((0, 0), (0, _CPAD - _CDIM)))
    w_out_t_pad = jnp.pad(w_out_t, ((0, _CPAD - _CDIM), (0, 0)))

    ind = _compute_indices(x2d, w_in_t, b_in2d, emb_t)
    q = _sc_gather(embed_pad, ind)
    out = _project_out(q, w_out_t_pad, b_out2d)
    return out.reshape(_B, _T, _DIM)


# speedup vs baseline: 1.0291x; 1.0216x over previous
import functools

import jax
import jax.numpy as jnp
from jax import lax
from jax.experimental import pallas as pl
from jax.experimental.pallas import tpu as pltpu
from jax.experimental.pallas import tpu_sc as plsc

_B, _T, _DIM, _CDIM, _K = 16, 2048, 512, 64, 1024
_CPAD = 128
_ROWS = _B * _T
_RB = 512
_NBLK = _ROWS // _RB


def _argmax_body(x_ref, wint_ref, bin_ref, embt_ref, ind_ref):
    z = jnp.dot(x_ref[...], wint_ref[...],
                preferred_element_type=jnp.float32) + bin_ref[...]
    et = embt_ref[...]
    esq = jnp.sum(et * et, axis=0, keepdims=True)
    fe = jnp.dot(z, et, preferred_element_type=jnp.float32)
    flatsq = jnp.sum(z * z, axis=1, keepdims=True)
    a = flatsq - 2.0 * fe + esq
    ind_ref[0, 0, :] = jnp.argmin(a, axis=1).astype(jnp.int32)


def _compute_indices(x2d, w_in_t, b_in2d, emb_t):
    ind3 = pl.pallas_call(
        _argmax_body,
        grid=(_NBLK,),
        in_specs=[
            pl.BlockSpec((_RB, _DIM), lambda i: (i, 0)),
            pl.BlockSpec((_DIM, _CDIM), lambda i: (0, 0)),
            pl.BlockSpec((1, _CDIM), lambda i: (0, 0)),
            pl.BlockSpec((_CDIM, _K), lambda i: (0, 0)),
        ],
        out_specs=pl.BlockSpec((1, 1, _RB), lambda i: (i, 0, 0)),
        out_shape=jax.ShapeDtypeStruct((_NBLK, 1, _RB), jnp.int32),
        compiler_params=pltpu.CompilerParams(
            dimension_semantics=("arbitrary",)),
    )(x2d, w_in_t, b_in2d, emb_t)
    return ind3.reshape(_ROWS)


_NC = 2
_NS = 16
_NW = _NC * _NS
_BPW = _ROWS // _NW
_CH = 128
_NCH = _BPW // _CH


def _sc_gather_body(emb_hbm, idx_hbm, q_hbm, idx_v, buf0, buf1, g0, g1):
    wid = lax.axis_index("s") * _NC + lax.axis_index("c")
    base = wid * _BPW
    pltpu.sync_copy(idx_hbm.at[wid], idx_v)
    pltpu.async_copy(emb_hbm.at[idx_v.at[0]], buf0, g0)

    def body(i, _):
        c0 = 2 * i
        pltpu.async_copy(emb_hbm.at[idx_v.at[c0 + 1]], buf1, g1)
        pltpu.make_async_copy(emb_hbm.at[idx_v.at[c0]], buf0, g0).wait()
        pltpu.sync_copy(buf0, q_hbm.at[pl.ds(base + c0 * _CH, _CH)])

        @pl.when(i < _NCH // 2 - 1)
        def _():
            pltpu.async_copy(emb_hbm.at[idx_v.at[c0 + 2]], buf0, g0)

        pltpu.make_async_copy(emb_hbm.at[idx_v.at[c0 + 1]], buf1, g1).wait()
        pltpu.sync_copy(buf1, q_hbm.at[pl.ds(base + (c0 + 1) * _CH, _CH)])
        return 0

    lax.fori_loop(0, _NCH // 2, body, 0)


def _sc_gather(embed_pad, ind3):
    mesh = plsc.VectorSubcoreMesh(core_axis_name="c", subcore_axis_name="s")
    k = functools.partial(
        pl.kernel,
        mesh=mesh,
        out_type=jax.ShapeDtypeStruct((_ROWS, _CPAD), jnp.float32),
        scratch_types=[
            pltpu.VMEM((_NCH, _CH), jnp.int32),
            pltpu.VMEM((_CH, _CPAD), jnp.float32),
            pltpu.VMEM((_CH, _CPAD), jnp.float32),
            pltpu.SemaphoreType.DMA,
            pltpu.SemaphoreType.DMA,
        ],
    )(_sc_gather_body)
    return k(embed_pad, ind3)


def _proj_out_body(q_ref, woutt_ref, bout_ref, out_ref):
    out_ref[...] = jnp.dot(q_ref[...], woutt_ref[...],
                           preferred_element_type=jnp.float32) + bout_ref[...]


def _project_out(q, w_out_t, b_out2d):
    return pl.pallas_call(
        _proj_out_body,
        grid=(_NBLK,),
        in_specs=[
            pl.BlockSpec((_RB, _CPAD), lambda i: (i, 0)),
            pl.BlockSpec((_CPAD, _DIM), lambda i: (0, 0)),
            pl.BlockSpec((1, _DIM), lambda i: (0, 0)),
        ],
        out_specs=pl.BlockSpec((_RB, _DIM), lambda i: (i, 0)),
        out_shape=jax.ShapeDtypeStruct((_ROWS, _DIM), jnp.float32),
        compiler_params=pltpu.CompilerParams(
            dimension_semantics=("arbitrary",)),
    )(q, w_out_t, b_out2d)


def kernel(x, W_in, b_in, W_out, b_out, embed):
    x2d = x.reshape(_ROWS, _DIM)
    w_in_t = W_in.T
    emb_t = embed.T
    b_in2d = b_in.reshape(1, _CDIM)
    w_out_t = W_out.T
    b_out2d = b_out.reshape(1, _DIM)

    embed_pad = jnp.pad(embed, ((0, 0), (0, _CPAD - _CDIM)))
    w_out_t_pad = jnp.pad(w_out_t, ((0, _CPAD - _CDIM), (0, 0)))

    ind = _compute_indices(x2d, w_in_t, b_in2d, emb_t)
    q = _sc_gather(embed_pad, ind.reshape(_NW, _NCH, _CH))
    out = _project_out(q, w_out_t_pad, b_out2d)
    return out.reshape(_B, _T, _DIM)
